# baseline (speedup 1.0000x reference)
import functools
import jax
import jax.numpy as jnp
from jax import lax
from jax.experimental import pallas as pl
from jax.experimental.pallas import tpu as pltpu
from jax.experimental.pallas import tpu_sc as plsc

N = 10000
D = 128
E = 320000
NR = 10240
NC = 2
NS = 16
NW = NC * NS
EB = 128
NB0 = 56
NB1 = 104
NBMX = 104
B0 = NS * NB0
TBAT = NS * (NB0 + NB1)
EPAD = TBAT * EB
RPT = NR // NS

_mesh = plsc.VectorSubcoreMesh(core_axis_name="c", subcore_axis_name="s")


@functools.partial(
    pl.kernel,
    out_type=jax.ShapeDtypeStruct((NC, NR, D), jnp.float32),
    mesh=_mesh,
    scratch_types=[
        pltpu.VMEM((NBMX, EB), jnp.int32),
        pltpu.VMEM((NBMX, EB), jnp.int32),
        pltpu.VMEM((EB, D), jnp.float32),
        pltpu.VMEM_SHARED((NR, D), jnp.float32),
        pltpu.SemaphoreType.DMA,
    ],
)
def _sc_agg(h_hbm, src_hbm, dst_hbm, zero_hbm, out_hbm,
            src_v, dst_v, rows_v, agg_sh, sem):
    cid = lax.axis_index("c")
    sid = lax.axis_index("s")
    my_nb = jnp.where(cid == 0, NB0, NB1)
    row0 = pl.multiple_of(
        jnp.where(cid == 0, sid * NB0, B0 + sid * NB1), 8)
    pltpu.sync_copy(zero_hbm, agg_sh.at[pl.ds(sid * RPT, RPT)])

    @pl.when(cid == 0)
    def _stage0():
        pltpu.sync_copy(src_hbm.at[pl.ds(row0, NB0)], src_v.at[pl.ds(0, NB0)])
        pltpu.sync_copy(dst_hbm.at[pl.ds(row0, NB0)], dst_v.at[pl.ds(0, NB0)])

    @pl.when(cid == 1)
    def _stage1():
        pltpu.sync_copy(src_hbm.at[pl.ds(row0, NB1)], src_v.at[pl.ds(0, NB1)])
        pltpu.sync_copy(dst_hbm.at[pl.ds(row0, NB1)], dst_v.at[pl.ds(0, NB1)])

    plsc.subcore_barrier()

    def body(b, carry):
        pltpu.async_copy(h_hbm.at[src_v.at[b]], rows_v, sem).wait()
        pltpu.sync_copy(rows_v, agg_sh.at[dst_v.at[b]], add=True)
        return carry

    lax.fori_loop(0, my_nb, body, 0)
    plsc.subcore_barrier()
    pltpu.sync_copy(agg_sh.at[pl.ds(sid * RPT, RPT)],
                    out_hbm.at[cid, pl.ds(sid * RPT, RPT)])


RB = 1000


def _mid_layer_body(h_ref, a0_ref, a1_ref, res_ref, w1_ref, b1_ref, w2_ref,
                    b2_ref, g_ref, be_ref, m_ref, v_ref, res_o, h_o):
    z = h_ref[...] + a0_ref[0] + a1_ref[0]
    u = jnp.maximum(jnp.dot(z, w1_ref[...], preferred_element_type=jnp.float32)
                    + b1_ref[...], 0.0)
    v = jnp.dot(u, w2_ref[...], preferred_element_type=jnp.float32) + b2_ref[...]
    scale = g_ref[...] * lax.rsqrt(v_ref[...] + 1e-5)
    h1 = (v - m_ref[...]) * scale + be_ref[...] + res_ref[...]
    res_o[...] = h1
    h_o[...] = jnp.maximum(h1, 0.0)


def _final_layer_body(h_ref, a0_ref, a1_ref, w1_ref, b1_ref, w2_ref, b2_ref,
                      out_ref):
    z = h_ref[...] + a0_ref[0] + a1_ref[0]
    u = jnp.maximum(jnp.dot(z, w1_ref[...], preferred_element_type=jnp.float32)
                    + b1_ref[...], 0.0)
    out_ref[...] = (jnp.dot(u, w2_ref[...], preferred_element_type=jnp.float32)
                    + b2_ref[...])


_row_spec = pl.BlockSpec((RB, D), lambda i: (i, 0))
_a0_spec = pl.BlockSpec((1, RB, D), lambda i: (0, i, 0))
_a1_spec = pl.BlockSpec((1, RB, D), lambda i: (1, i, 0))
_w_spec = pl.BlockSpec((D, D), lambda i: (0, 0))
_v_spec = pl.BlockSpec((1, D), lambda i: (0, 0))


def _tc_mid_layer(h, agg, res, w1, b1, w2, b2, g, be, m, v):
    return pl.pallas_call(
        _mid_layer_body,
        grid=(N // RB,),
        in_specs=[_row_spec, _a0_spec, _a1_spec, _row_spec, _w_spec, _v_spec,
                  _w_spec, _v_spec, _v_spec, _v_spec, _v_spec, _v_spec],
        out_specs=[_row_spec, _row_spec],
        out_shape=[jax.ShapeDtypeStruct((N, D), jnp.float32),
                   jax.ShapeDtypeStruct((N, D), jnp.float32)],
    )(h, agg, agg, res, w1, b1.reshape(1, D), w2, b2.reshape(1, D),
      g.reshape(1, D), be.reshape(1, D), m.reshape(1, D), v.reshape(1, D))


def _tc_final_layer(h, agg, w1, b1, w2, b2):
    return pl.pallas_call(
        _final_layer_body,
        grid=(N // RB,),
        in_specs=[_row_spec, _a0_spec, _a1_spec, _w_spec, _v_spec, _w_spec,
                  _v_spec],
        out_specs=_row_spec,
        out_shape=jax.ShapeDtypeStruct((N, D), jnp.float32),
    )(h, agg, agg, w1, b1.reshape(1, D), w2, b2.reshape(1, D))


def kernel(x, edge_index, c0_w1, c0_b1, c0_w2, c0_b2, c1_w1, c1_b1, c1_w2,
           c1_b2, c2_w1, c2_b1, c2_w2, c2_b2, bn0_gamma, bn0_beta, bn0_mean,
           bn0_var, bn1_gamma, bn1_beta, bn1_mean, bn1_var):
    src = edge_index[0].astype(jnp.int32)
    dst = edge_index[1].astype(jnp.int32)
    npad = EPAD - E
    srcr = jnp.concatenate([src, jnp.zeros((npad,), jnp.int32)]).reshape(
        TBAT, EB)
    dstr = jnp.concatenate([dst, jnp.full((npad,), NR - 1, jnp.int32)]).reshape(
        TBAT, EB)
    zero_rows = jnp.zeros((RPT, D), jnp.float32)

    agg = _sc_agg(x, srcr, dstr, zero_rows)
    res, h = _tc_mid_layer(x, agg, x, c0_w1, c0_b1, c0_w2, c0_b2,
                           bn0_gamma, bn0_beta, bn0_mean, bn0_var)
    agg = _sc_agg(h, srcr, dstr, zero_rows)
    res, h = _tc_mid_layer(h, agg, res, c1_w1, c1_b1, c1_w2, c1_b2,
                           bn1_gamma, bn1_beta, bn1_mean, bn1_var)
    agg = _sc_agg(h, srcr, dstr, zero_rows)
    return _tc_final_layer(h, agg, c2_w1, c2_b1, c2_w2, c2_b2)


# speedup vs baseline: 1.5752x; 1.5752x over previous
import functools
import jax
import jax.numpy as jnp
from jax import lax
from jax.experimental import pallas as pl
from jax.experimental.pallas import tpu as pltpu
from jax.experimental.pallas import tpu_sc as plsc

N = 10000
D = 128
E = 320000
NR = 10240
NC = 2
NS = 16
NW = NC * NS
EB = 128
NBATCH = 79
EPW = EB * NBATCH
EPAD = EPW * NW
RPT = NR // NS

_mesh = plsc.VectorSubcoreMesh(core_axis_name="c", subcore_axis_name="s")


@functools.partial(
    pl.kernel,
    out_type=jax.ShapeDtypeStruct((NC, NR, D), jnp.float32),
    mesh=_mesh,
    scratch_types=[
        pltpu.VMEM((NBATCH, EB), jnp.int32),
        pltpu.VMEM((NBATCH, EB), jnp.int32),
        pltpu.VMEM((EB, D), jnp.float32),
        pltpu.VMEM_SHARED((NR, D), jnp.float32),
        pltpu.SemaphoreType.DMA,
    ],
)
def _sc_agg(h_hbm, src_hbm, dst_hbm, zero_hbm, out_hbm,
            src_v, dst_v, rows_v, agg_sh, sem):
    cid = lax.axis_index("c")
    sid = lax.axis_index("s")
    wid = sid * NC + cid
    pltpu.sync_copy(zero_hbm, agg_sh.at[pl.ds(sid * RPT, RPT)])
    pltpu.sync_copy(src_hbm.at[wid], src_v)
    pltpu.sync_copy(dst_hbm.at[wid], dst_v)
    plsc.subcore_barrier()

    def body(b, carry):
        pltpu.async_copy(h_hbm.at[src_v.at[b]], rows_v, sem).wait()
        pltpu.sync_copy(rows_v, agg_sh.at[dst_v.at[b]], add=True)
        return carry

    lax.fori_loop(0, NBATCH, body, 0)
    plsc.subcore_barrier()
    pltpu.sync_copy(agg_sh.at[pl.ds(sid * RPT, RPT)],
                    out_hbm.at[cid, pl.ds(sid * RPT, RPT)])


RB = 1000


def _mid_layer_body(h_ref, a0_ref, a1_ref, res_ref, w1_ref, b1_ref, w2_ref,
                    b2_ref, g_ref, be_ref, m_ref, v_ref, res_o, h_o):
    z = h_ref[...] + a0_ref[0] + a1_ref[0]
    u = jnp.maximum(jnp.dot(z, w1_ref[...], preferred_element_type=jnp.float32)
                    + b1_ref[...], 0.0)
    v = jnp.dot(u, w2_ref[...], preferred_element_type=jnp.float32) + b2_ref[...]
    scale = g_ref[...] * lax.rsqrt(v_ref[...] + 1e-5)
    h1 = (v - m_ref[...]) * scale + be_ref[...] + res_ref[...]
    res_o[...] = h1
    h_o[...] = jnp.maximum(h1, 0.0)


def _final_layer_body(h_ref, a0_ref, a1_ref, w1_ref, b1_ref, w2_ref, b2_ref,
                      out_ref):
    z = h_ref[...] + a0_ref[0] + a1_ref[0]
    u = jnp.maximum(jnp.dot(z, w1_ref[...], preferred_element_type=jnp.float32)
                    + b1_ref[...], 0.0)
    out_ref[...] = (jnp.dot(u, w2_ref[...], preferred_element_type=jnp.float32)
                    + b2_ref[...])


_row_spec = pl.BlockSpec((RB, D), lambda i: (i, 0))
_a0_spec = pl.BlockSpec((1, RB, D), lambda i: (0, i, 0))
_a1_spec = pl.BlockSpec((1, RB, D), lambda i: (1, i, 0))
_w_spec = pl.BlockSpec((D, D), lambda i: (0, 0))
_v_spec = pl.BlockSpec((1, D), lambda i: (0, 0))


def _tc_mid_layer(h, agg, res, w1, b1, w2, b2, g, be, m, v):
    return pl.pallas_call(
        _mid_layer_body,
        grid=(N // RB,),
        in_specs=[_row_spec, _a0_spec, _a1_spec, _row_spec, _w_spec, _v_spec,
                  _w_spec, _v_spec, _v_spec, _v_spec, _v_spec, _v_spec],
        out_specs=[_row_spec, _row_spec],
        out_shape=[jax.ShapeDtypeStruct((N, D), jnp.float32),
                   jax.ShapeDtypeStruct((N, D), jnp.float32)],
    )(h, agg, agg, res, w1, b1.reshape(1, D), w2, b2.reshape(1, D),
      g.reshape(1, D), be.reshape(1, D), m.reshape(1, D), v.reshape(1, D))


def _tc_final_layer(h, agg, w1, b1, w2, b2):
    return pl.pallas_call(
        _final_layer_body,
        grid=(N // RB,),
        in_specs=[_row_spec, _a0_spec, _a1_spec, _w_spec, _v_spec, _w_spec,
                  _v_spec],
        out_specs=_row_spec,
        out_shape=jax.ShapeDtypeStruct((N, D), jnp.float32),
    )(h, agg, agg, w1, b1.reshape(1, D), w2, b2.reshape(1, D))


def kernel(x, edge_index, c0_w1, c0_b1, c0_w2, c0_b2, c1_w1, c1_b1, c1_w2,
           c1_b2, c2_w1, c2_b1, c2_w2, c2_b2, bn0_gamma, bn0_beta, bn0_mean,
           bn0_var, bn1_gamma, bn1_beta, bn1_mean, bn1_var):
    src = edge_index[0].astype(jnp.int32)
    dst = edge_index[1].astype(jnp.int32)
    npad = EPAD - E
    srcr = jnp.concatenate([src, jnp.zeros((npad,), jnp.int32)]).reshape(
        NW, NBATCH, EB)
    junk = N + (jnp.arange(npad, dtype=jnp.int32) % (NR - N))
    dstr = jnp.concatenate([dst, junk]).reshape(NW, NBATCH, EB)
    zero_rows = jnp.zeros((RPT, D), jnp.float32)

    agg = _sc_agg(x, srcr, dstr, zero_rows)
    res, h = _tc_mid_layer(x, agg, x, c0_w1, c0_b1, c0_w2, c0_b2,
                           bn0_gamma, bn0_beta, bn0_mean, bn0_var)
    agg = _sc_agg(h, srcr, dstr, zero_rows)
    res, h = _tc_mid_layer(h, agg, res, c1_w1, c1_b1, c1_w2, c1_b2,
                           bn1_gamma, bn1_beta, bn1_mean, bn1_var)
    agg = _sc_agg(h, srcr, dstr, zero_rows)
    return _tc_final_layer(h, agg, c2_w1, c2_b1, c2_w2, c2_b2)
